# baseline (speedup 1.0000x reference)
import functools

import jax
import jax.numpy as jnp
from jax.experimental import pallas as pl
from jax.experimental.pallas import tpu as pltpu

B, N, T = 2, 4096, 4
FIN, FOUT = 64, 32
COLS = FOUT * T

TN = 512
XC = 1024

_SLOPE = 0.01


def _lrelu(v):
    return jnp.where(v >= 0, v, _SLOPE * v)


def _fused_body(adj_ref, xt_ref, w_ref, b_ref, y0_ref, y_ref, h12s, g2s):
    phase = pl.program_id(0)
    b = pl.program_id(1)
    n = pl.program_id(2)

    @pl.when((phase == 0) & (n == 0))
    def _():
        w = w_ref[...].astype(jnp.bfloat16)
        bias = b_ref[0][None, :]
        for i in range(N // XC):
            xc = xt_ref[0, i * XC:(i + 1) * XC, :].astype(jnp.bfloat16)
            d = jnp.dot(xc, w, preferred_element_type=jnp.float32) + bias
            y0_ref[0, i * XC:(i + 1) * XC, :] = _lrelu(d[:, :COLS]).astype(jnp.bfloat16)
            h12s[i * XC:(i + 1) * XC, :] = d[:, COLS:].astype(jnp.bfloat16)

    a = adj_ref[0].astype(jnp.bfloat16)

    @pl.when(phase == 0)
    def _():
        pp = jnp.dot(a, h12s[...], preferred_element_type=jnp.float32)
        y_ref[0, 0] = _lrelu(pp[:, :COLS]).astype(jnp.bfloat16)
        g2s[b, pl.ds(n * TN, TN), :] = pp[:, COLS:].astype(jnp.bfloat16)

    @pl.when(phase == 1)
    def _():
        p = jnp.dot(a, g2s[b, :, :], preferred_element_type=jnp.float32)
        y_ref[0, 0] = _lrelu(p).astype(jnp.bfloat16)


def _block_diag_t(w):
    z = jnp.zeros((T * FIN, T * FOUT), w.dtype)
    for t in range(T):
        z = z.at[t * FIN:(t + 1) * FIN, t * FOUT:(t + 1) * FOUT].set(w)
    return z


@functools.partial(jax.jit)
def _impl(x, adj, W0, b0, W1, b1, W2, b2):
    xt = x.transpose(0, 2, 3, 1).reshape(B, N, T * FIN)
    wall = jnp.concatenate(
        [_block_diag_t(W0), _block_diag_t(W1), _block_diag_t(W2)], axis=1)
    ball = jnp.concatenate(
        [jnp.tile(b0, T), jnp.tile(b1, T), jnp.tile(b2, T)]).reshape(1, 3 * COLS)

    y0, ys = pl.pallas_call(
        _fused_body,
        grid=(2, B, N // TN),
        in_specs=[
            pl.BlockSpec((1, TN, N), lambda p, b, n: (b, n, 0)),
            pl.BlockSpec((1, N, T * FIN), lambda p, b, n: (b, 0, 0)),
            pl.BlockSpec((T * FIN, 3 * COLS), lambda p, b, n: (0, 0)),
            pl.BlockSpec((1, 3 * COLS), lambda p, b, n: (0, 0)),
        ],
        out_specs=[
            pl.BlockSpec((1, N, COLS), lambda p, b, n: (b * (1 - p) + B * p, 0, 0)),
            pl.BlockSpec((1, 1, TN, COLS), lambda p, b, n: (p, b, n, 0)),
        ],
        out_shape=[
            jax.ShapeDtypeStruct((B + 1, N, COLS), jnp.bfloat16),
            jax.ShapeDtypeStruct((2, B, N, COLS), jnp.bfloat16),
        ],
        scratch_shapes=[
            pltpu.VMEM((N, 2 * COLS), jnp.bfloat16),
            pltpu.VMEM((B, N, COLS), jnp.bfloat16),
        ],
        compiler_params=pltpu.CompilerParams(
            dimension_semantics=("arbitrary", "arbitrary", "arbitrary")),
    )(adj, xt, wall, ball)
    y0 = y0[:B]
    y1 = ys[0]
    y2 = ys[1]

    def unpack(y):
        return y.reshape(B, N, T, FOUT).transpose(0, 3, 1, 2)

    out = jnp.concatenate([unpack(y0), unpack(y1), unpack(y2)], axis=1)
    return out.astype(jnp.float32)


def kernel(x, adj, W0, b0, W1, b1, W2, b2):
    return _impl(x, adj, W0, b0, W1, b1, W2, b2)


# speedup vs baseline: 1.0400x; 1.0400x over previous
import functools

import jax
import jax.numpy as jnp
from jax.experimental import pallas as pl
from jax.experimental.pallas import tpu as pltpu

B, N, T = 2, 4096, 4
FIN, FOUT = 64, 32
COLS = FOUT * T

TN = 1024
XC = 1024

_SLOPE = 0.01


def _lrelu(v):
    return jnp.where(v >= 0, v, _SLOPE * v)


def _fused_body(adj_ref, xt_ref, w_ref, b_ref, y0_ref, y_ref, h12s, g2s):
    phase = pl.program_id(0)
    b = pl.program_id(1)
    n = pl.program_id(2)

    @pl.when((phase == 0) & (n == 0))
    def _():
        w = w_ref[...].astype(jnp.bfloat16)
        bias = b_ref[0][None, :]
        for i in range(N // XC):
            xc = xt_ref[0, i * XC:(i + 1) * XC, :].astype(jnp.bfloat16)
            d = jnp.dot(xc, w, preferred_element_type=jnp.float32) + bias
            y0_ref[0, i * XC:(i + 1) * XC, :] = _lrelu(d[:, :COLS]).astype(jnp.bfloat16)
            h12s[i * XC:(i + 1) * XC, :] = d[:, COLS:].astype(jnp.bfloat16)

    a = adj_ref[0].astype(jnp.bfloat16)

    @pl.when(phase == 0)
    def _():
        pp = jnp.dot(a, h12s[...], preferred_element_type=jnp.float32)
        y_ref[0, 0] = _lrelu(pp[:, :COLS]).astype(jnp.bfloat16)
        g2s[b, pl.ds(n * TN, TN), :] = pp[:, COLS:].astype(jnp.bfloat16)

    @pl.when(phase == 1)
    def _():
        p = jnp.dot(a, g2s[b, :, :], preferred_element_type=jnp.float32)
        y_ref[0, 0] = _lrelu(p).astype(jnp.bfloat16)


def _block_diag_t(w):
    z = jnp.zeros((T * FIN, T * FOUT), w.dtype)
    for t in range(T):
        z = z.at[t * FIN:(t + 1) * FIN, t * FOUT:(t + 1) * FOUT].set(w)
    return z


@functools.partial(jax.jit)
def _impl(x, adj, W0, b0, W1, b1, W2, b2):
    xt = x.transpose(0, 2, 3, 1).reshape(B, N, T * FIN)
    wall = jnp.concatenate(
        [_block_diag_t(W0), _block_diag_t(W1), _block_diag_t(W2)], axis=1)
    ball = jnp.concatenate(
        [jnp.tile(b0, T), jnp.tile(b1, T), jnp.tile(b2, T)]).reshape(1, 3 * COLS)

    y0, ys = pl.pallas_call(
        _fused_body,
        grid=(2, B, N // TN),
        in_specs=[
            pl.BlockSpec((1, TN, N), lambda p, b, n: (b, n, 0)),
            pl.BlockSpec((1, N, T * FIN),
                         lambda p, b, n: (b * (1 - p) + (B - 1) * p, 0, 0)),
            pl.BlockSpec((T * FIN, 3 * COLS), lambda p, b, n: (0, 0)),
            pl.BlockSpec((1, 3 * COLS), lambda p, b, n: (0, 0)),
        ],
        out_specs=[
            pl.BlockSpec((1, N, COLS), lambda p, b, n: (b * (1 - p) + B * p, 0, 0)),
            pl.BlockSpec((1, 1, TN, COLS), lambda p, b, n: (p, b, n, 0)),
        ],
        out_shape=[
            jax.ShapeDtypeStruct((B + 1, N, COLS), jnp.bfloat16),
            jax.ShapeDtypeStruct((2, B, N, COLS), jnp.bfloat16),
        ],
        scratch_shapes=[
            pltpu.VMEM((N, 2 * COLS), jnp.bfloat16),
            pltpu.VMEM((B, N, COLS), jnp.bfloat16),
        ],
        compiler_params=pltpu.CompilerParams(
            dimension_semantics=("arbitrary", "arbitrary", "arbitrary")),
    )(adj, xt, wall, ball)
    y0 = y0[:B]
    y1 = ys[0]
    y2 = ys[1]

    def unpack(y):
        return y.reshape(B, N, T, FOUT).transpose(0, 3, 1, 2)

    out = jnp.concatenate([unpack(y0), unpack(y1), unpack(y2)], axis=1)
    return out.astype(jnp.float32)


def kernel(x, adj, W0, b0, W1, b1, W2, b2):
    return _impl(x, adj, W0, b0, W1, b1, W2, b2)
